# baseline (speedup 1.0000x reference)
import functools

import jax
import jax.numpy as jnp
from jax import lax
from jax.experimental import pallas as pl
from jax.experimental.pallas import tpu as pltpu
from jax.experimental.pallas import tpu_sc as plsc

NC = 2
NS = 16
NW = NC * NS
L = 16
CH = 128
PACK = 4


def _make_sc_kernel(B, D, b_per_w, n_ch):
    mesh = plsc.VectorSubcoreMesh(core_axis_name="c", subcore_axis_name="s")

    @functools.partial(
        pl.kernel,
        out_type=jax.ShapeDtypeStruct((NW, b_per_w), jnp.float32),
        mesh=mesh,
        compiler_params=pltpu.CompilerParams(needs_layout_passes=False),
        scratch_types=[
            pltpu.VMEM((b_per_w,), jnp.int32),
            pltpu.VMEM((b_per_w,), jnp.int32),
            pltpu.VMEM((b_per_w,), jnp.int32),
            pltpu.VMEM((b_per_w,), jnp.int32),
            pltpu.VMEM((2, CH, PACK * D), jnp.float32),
            pltpu.VMEM((2, CH, PACK * D), jnp.float32),
            pltpu.VMEM((b_per_w,), jnp.float32),
            pltpu.SemaphoreType.DMA,
            pltpu.SemaphoreType.DMA,
        ],
    )
    def run(user_hbm, article_hbm, ctab_hbm, atab_hbm, out_hbm,
            idx_u, idx_a, pidx_u, pidx_a, cbuf, abuf, out_v, sem_c, sem_a):
        wid = lax.axis_index("s") * NC + lax.axis_index("c")
        pltpu.sync_copy(user_hbm.at[wid], idx_u)
        pltpu.sync_copy(article_hbm.at[wid], idx_a)
        for v in range(b_per_w // L):
            sl = pl.ds(v * L, L)
            pidx_u[sl] = lax.shift_right_logical(idx_u[sl], 2)
            pidx_a[sl] = lax.shift_right_logical(idx_a[sl], 2)

        def fire(j):
            b = j % 2
            hc = pltpu.async_copy(
                ctab_hbm.at[pidx_u.at[pl.ds(j * CH, CH)]], cbuf.at[b], sem_c)
            ha = pltpu.async_copy(
                atab_hbm.at[pidx_a.at[pl.ds(j * CH, CH)]], abuf.at[b], sem_a)
            return hc, ha

        handles = [None] * n_ch
        handles[0] = fire(0)
        for j in range(n_ch):
            if j + 1 < n_ch:
                handles[j + 1] = fire(j + 1)
            handles[j][0].wait()
            handles[j][1].wait()
            b = j % 2
            cb = cbuf.at[b]
            ab = abuf.at[b]

            def group(g, carry, j=j, cb=cb, ab=ab):
                rows = g * L + lax.iota(jnp.int32, L)
                u = idx_u[pl.ds(j * CH + g * L, L)]
                a = idx_a[pl.ds(j * CH + g * L, L)]
                ucol = (u & 3) * D
                acol = (a & 3) * D
                acc = jnp.zeros((L,), jnp.float32)
                for d in range(D):
                    cv = plsc.load_gather(cb, [rows, ucol + d])
                    av = plsc.load_gather(ab, [rows, acol + d])
                    acc = acc + cv * av
                out_v[pl.ds(j * CH + g * L, L)] = acc
                return carry

            lax.fori_loop(0, CH // L, group, 0)
        pltpu.sync_copy(out_v, out_hbm.at[wid])

    return run


@jax.jit
def kernel(user, article, customer_table, article_table):
    B = user.shape[0]
    D = customer_table.shape[1]
    b_per_w = B // NW
    n_ch = b_per_w // CH
    user_r = user.reshape(NW, b_per_w)
    article_r = article.reshape(NW, b_per_w)
    ctab_p = customer_table.reshape(-1, PACK * D)
    atab_p = article_table.reshape(-1, PACK * D)
    run = _make_sc_kernel(B, D, b_per_w, n_ch)
    out = run(user_r, article_r, ctab_p, atab_p)
    return out.reshape(B, 1)


# speedup vs baseline: 1.0040x; 1.0040x over previous
import functools

import jax
import jax.numpy as jnp
from jax import lax
from jax.experimental import pallas as pl
from jax.experimental.pallas import tpu as pltpu
from jax.experimental.pallas import tpu_sc as plsc

NC = 2
NS = 16
NW = NC * NS
L = 16


def _make_sc_kernel(D, b_per_w):
    mesh = plsc.VectorSubcoreMesh(core_axis_name="c", subcore_axis_name="s")

    @functools.partial(
        pl.kernel,
        out_type=jax.ShapeDtypeStruct((NW, b_per_w), jnp.float32),
        mesh=mesh,
        compiler_params=pltpu.CompilerParams(
            needs_layout_passes=False, use_tc_tiling_on_sc=False),
        scratch_types=[
            pltpu.VMEM((b_per_w,), jnp.int32),
            pltpu.VMEM((b_per_w,), jnp.int32),
            pltpu.VMEM((b_per_w, D), jnp.float32),
            pltpu.VMEM((b_per_w, D), jnp.float32),
            pltpu.VMEM((b_per_w,), jnp.float32),
            pltpu.SemaphoreType.DMA,
            pltpu.SemaphoreType.DMA,
        ],
    )
    def run(user_hbm, article_hbm, ctab_hbm, atab_hbm, out_hbm,
            idx_u, idx_a, crows, arows, out_v, sem_c, sem_a):
        wid = lax.axis_index("s") * NC + lax.axis_index("c")
        base = wid * b_per_w
        pltpu.sync_copy(user_hbm.at[pl.ds(base, b_per_w)], idx_u)
        pltpu.sync_copy(article_hbm.at[pl.ds(base, b_per_w)], idx_a)

        cp_c = pltpu.async_copy(ctab_hbm.at[idx_u], crows, sem_c)
        cp_a = pltpu.async_copy(atab_hbm.at[idx_a], arows, sem_a)
        cp_c.wait()
        cp_a.wait()

        lanes = lax.broadcasted_iota(jnp.int32, (L,), 0)

        def dot(i, carry):
            rows = lanes + i * L
            acc = None
            for d in range(D):
                col = jnp.full((L,), d, jnp.int32)
                p = (plsc.load_gather(crows, [rows, col])
                     * plsc.load_gather(arows, [rows, col]))
                acc = p if acc is None else acc + p
            out_v[pl.ds(i * L, L)] = acc
            return carry

        lax.fori_loop(0, b_per_w // L, dot, 0)
        pltpu.sync_copy(out_v, out_hbm.at[wid])

    return run


@jax.jit
def kernel(user, article, customer_table, article_table):
    B = user.shape[0]
    D = customer_table.shape[1]
    b_per_w = B // NW
    run = _make_sc_kernel(D, b_per_w)
    out = run(user.reshape(B), article.reshape(B),
              customer_table, article_table)
    return out.reshape(B, 1)
